# baseline (speedup 1.0000x reference)
import jax
import jax.numpy as jnp
from jax.experimental import pallas as pl
from jax.experimental.pallas import tpu as pltpu

_TOK = 16


def _pool_matrix(rows, pool):
    k = jax.lax.broadcasted_iota(jnp.int32, (rows, rows // pool), 0)
    v = jax.lax.broadcasted_iota(jnp.int32, (rows, rows // pool), 1)
    return jnp.where(k // pool == v, 1.0, 0.0).astype(jnp.float32)


def _roll_sub(x, m):
    r = x.reshape(16, 8, 128)
    r = jnp.concatenate([r[:, 8 - m:], r[:, :8 - m]], axis=1)
    return r.reshape(128, 128)


def _gran_xpose(x):
    s = jax.lax.broadcasted_iota(jnp.int32, (128, 128), 0) % 8
    g = jax.lax.broadcasted_iota(jnp.int32, (128, 128), 1) // _TOK
    for m in (4, 2, 1):
        xa = jnp.roll(_roll_sub(x, m), -_TOK * m, axis=1)
        xb = jnp.roll(_roll_sub(x, 8 - m), _TOK * m, axis=1)
        sm = (s & m) != 0
        gm = (g & m) != 0
        x = jnp.where(sm == gm, x, jnp.where(sm, xa, xb))
    return x


def _tokens(p):
    q = _gran_xpose(p).reshape(8, 2, 8, 128)
    return jnp.concatenate(
        [q[:, 0].reshape(64, 128), q[:, 1].reshape(64, 128)], axis=1)


def _ring(x):
    parts = [x[0:16]]
    for i in range(2, 6):
        parts.append(x[8 * i:8 * i + 2])
        parts.append(x[8 * i + 6:8 * i + 8])
    parts.append(x[48:64])
    return jnp.concatenate(parts, axis=0)


_IMGS_PER_PROG = 2


def _pool_ring(img, s_ref, pool):
    rows = img.shape[0]
    n = rows // 4
    m = _pool_matrix(rows, pool)
    s_ref[0:n, :] = jnp.dot(img[0:n], m, preferred_element_type=jnp.float32)
    s_ref[3 * n:4 * n, :] = jnp.dot(img[3 * n:4 * n], m,
                                    preferred_element_type=jnp.float32)
    mid = img[n:3 * n]
    q = n // pool
    s_ref[n:3 * n, 0:q] = jnp.dot(mid, m[:, 0:q],
                                  preferred_element_type=jnp.float32)
    s_ref[n:3 * n, 3 * q:4 * q] = jnp.dot(mid, m[:, 3 * q:4 * q],
                                          preferred_element_type=jnp.float32)


def _fov_kernel(img_ref, out_ref, s1_ref, s2_ref):
    for bb in range(_IMGS_PER_PROG):
        cols = []
        for c in range(3):
            p0 = img_ref[bb, c, 192:320, 192:320]
            _pool_ring(img_ref[bb, c, 128:384, 128:384], s1_ref, 2)
            p1 = (s1_ref[0::2, :] + s1_ref[1::2, :]) * 0.25
            _pool_ring(img_ref[bb, c], s2_ref, 4)
            p2 = (s2_ref[0::4, :] + s2_ref[1::4, :]
                  + s2_ref[2::4, :] + s2_ref[3::4, :]) * 0.0625
            cols.append(jnp.concatenate(
                [_tokens(p0), _ring(_tokens(p1)), _ring(_tokens(p2))],
                axis=0))
        out_ref[bb] = jnp.concatenate(cols, axis=1)


def kernel(images):
    B, C, H, W = images.shape
    g = _IMGS_PER_PROG
    out = pl.pallas_call(
        _fov_kernel,
        grid=(B // g,),
        in_specs=[pl.BlockSpec((g, C, H, W), lambda b: (b, 0, 0, 0))],
        out_specs=pl.BlockSpec((g, 160, C * 256), lambda b: (b, 0, 0)),
        out_shape=jax.ShapeDtypeStruct((B, 160, C * 256), jnp.float32),
        scratch_shapes=[pltpu.VMEM((256, 128), jnp.float32),
                        pltpu.VMEM((512, 128), jnp.float32)],
        compiler_params=pltpu.CompilerParams(
            dimension_semantics=("arbitrary",)),
    )(images)
    return out.reshape(B, 160, C, _TOK, _TOK)


# speedup vs baseline: 1.0888x; 1.0888x over previous
import jax
import jax.numpy as jnp
from jax.experimental import pallas as pl
from jax.experimental.pallas import tpu as pltpu

_TOK = 16


def _pool_matrix(rows, pool):
    k = jax.lax.broadcasted_iota(jnp.int32, (rows, rows // pool), 0)
    v = jax.lax.broadcasted_iota(jnp.int32, (rows, rows // pool), 1)
    return jnp.where(k // pool == v, 1.0, 0.0).astype(jnp.float32)


def _roll_sub(x, m):
    r = x.reshape(16, 8, 128)
    r = jnp.concatenate([r[:, 8 - m:], r[:, :8 - m]], axis=1)
    return r.reshape(128, 128)


def _gran_xpose(x):
    s = jax.lax.broadcasted_iota(jnp.int32, (128, 128), 0) % 8
    g = jax.lax.broadcasted_iota(jnp.int32, (128, 128), 1) // _TOK
    for m in (4, 2, 1):
        xa = jnp.roll(_roll_sub(x, m), -_TOK * m, axis=1)
        xb = jnp.roll(_roll_sub(x, 8 - m), _TOK * m, axis=1)
        sm = (s & m) != 0
        gm = (g & m) != 0
        x = jnp.where(sm == gm, x, jnp.where(sm, xa, xb))
    return x


def _tokens(p):
    q = _gran_xpose(p).reshape(8, 2, 8, 128)
    return jnp.concatenate(
        [q[:, 0].reshape(64, 128), q[:, 1].reshape(64, 128)], axis=1)


def _ring(x):
    parts = [x[0:16]]
    for i in range(2, 6):
        parts.append(x[8 * i:8 * i + 2])
        parts.append(x[8 * i + 6:8 * i + 8])
    parts.append(x[48:64])
    return jnp.concatenate(parts, axis=0)


_IMGS_PER_PROG = 2


def _fov_kernel(img_ref, out_ref, s1_ref, s2_ref):
    for bb in range(_IMGS_PER_PROG):
        cols = []
        for c in range(3):
            p0 = img_ref[bb, c, 192:320, 192:320]
            s1_ref[...] = jnp.dot(img_ref[bb, c, 128:384, 128:384],
                                  _pool_matrix(256, 2),
                                  preferred_element_type=jnp.float32)
            p1 = (s1_ref[0::2, :] + s1_ref[1::2, :]) * 0.25
            s2_ref[...] = jnp.dot(img_ref[bb, c], _pool_matrix(512, 4),
                                  preferred_element_type=jnp.float32)
            p2 = (s2_ref[0::4, :] + s2_ref[1::4, :]
                  + s2_ref[2::4, :] + s2_ref[3::4, :]) * 0.0625
            cols.append(jnp.concatenate(
                [_tokens(p0), _ring(_tokens(p1)), _ring(_tokens(p2))],
                axis=0))
        out_ref[bb] = jnp.concatenate(cols, axis=1)


def kernel(images):
    B, C, H, W = images.shape
    g = _IMGS_PER_PROG
    out = pl.pallas_call(
        _fov_kernel,
        grid=(B // g,),
        in_specs=[pl.BlockSpec((g, C, H, W), lambda b: (b, 0, 0, 0))],
        out_specs=pl.BlockSpec((g, 160, C * 256), lambda b: (b, 0, 0)),
        out_shape=jax.ShapeDtypeStruct((B, 160, C * 256), jnp.float32),
        scratch_shapes=[pltpu.VMEM((256, 128), jnp.float32),
                        pltpu.VMEM((512, 128), jnp.float32)],
        compiler_params=pltpu.CompilerParams(
            dimension_semantics=("arbitrary",)),
    )(images)
    return out.reshape(B, 160, C, _TOK, _TOK)
